# Initial kernel scaffold; baseline (speedup 1.0000x reference)
import jax
import jax.numpy as jnp
from jax.experimental import pallas as pl


def kernel(h, edge_index, edge_type, r, W, b):
    raise NotImplementedError("write your pallas kernel here")


# speedup vs baseline: 2.7901x; 2.7901x over previous
import jax
import jax.numpy as jnp
from jax import lax
from jax.experimental import pallas as pl
from jax.experimental.pallas import tpu as pltpu
from jax.experimental.pallas import tpu_sc as plsc

N = 10000
E = 160000
D = 128
R = 4
OUT = 128

NC = 2
NS = 16
NW = NC * NS

NP = 10240
OFF_B = NP * R
OFF_H = 2 * NP * R

DEG_BASE = NP
DEG_ROWS = NP // D
ACC_ROWS = 10368
ROWS_PER_SUB = ACC_ROWS // NS
WB = 72
NB = ROWS_PER_SUB // WB

C = 64
EW = 5120
EP = NW * EW
NCHUNK = EW // C


def _proj_body(h_ref, rd_ref, rs_ref, a_ref, b_ref):
    hb = h_ref[...]
    a_ref[...] = jnp.dot(hb, rd_ref[...], preferred_element_type=jnp.float32)
    b_ref[...] = jnp.dot(hb, rs_ref[...], preferred_element_type=jnp.float32)


def _project(h_pad, rd, rs):
    bm = NP // 4
    return pl.pallas_call(
        _proj_body,
        grid=(4,),
        in_specs=[
            pl.BlockSpec((bm, D), lambda i: (i, 0)),
            pl.BlockSpec((D, R * D), lambda i: (0, 0)),
            pl.BlockSpec((D, R * D), lambda i: (0, 0)),
        ],
        out_specs=[
            pl.BlockSpec((bm, R * D), lambda i: (i, 0)),
            pl.BlockSpec((bm, R * D), lambda i: (i, 0)),
        ],
        out_shape=[
            jax.ShapeDtypeStruct((NP, R * D), jnp.float32),
            jax.ShapeDtypeStruct((NP, R * D), jnp.float32),
        ],
    )(h_pad, rd, rs)


def _edge_body(tbl_hbm, src_hbm, dst_hbm, et_hbm, out_hbm,
               isrc, idst, iet, ia, ib, id2,
               rowa, rowb, rowh, oh_v, wb, acc, sem):
    c = lax.axis_index("c")
    s = lax.axis_index("s")
    wid = s * NC + c

    zeros16 = jnp.zeros((16,), jnp.float32)
    iota16 = lax.iota(jnp.int32, 16)

    def _zero_wb(i, _):
        for k in range(D // 16):
            wb[i, pl.ds(k * 16, 16)] = zeros16
        return 0

    lax.fori_loop(0, WB, _zero_wb, 0)

    def _zero_oh(j, _):
        for k in range(D // 16):
            oh_v[j, pl.ds(k * 16, 16)] = zeros16
        return 0

    lax.fori_loop(0, C, _zero_oh, 0)

    base_row = s * ROWS_PER_SUB

    def _zero_acc(t, _):
        pltpu.sync_copy(wb, acc.at[pl.ds(base_row + t * WB, WB)])
        return 0

    lax.fori_loop(0, NB, _zero_acc, 0)
    plsc.subcore_barrier()

    ebase = wid * EW

    def _chunk(g, _):
        base = ebase + g * C
        pltpu.sync_copy(src_hbm.at[pl.ds(base, C)], isrc)
        pltpu.sync_copy(dst_hbm.at[pl.ds(base, C)], idst)
        pltpu.sync_copy(et_hbm.at[pl.ds(base, C)], iet)

        def _idx(k, _):
            sl = pl.ds(k * 16, 16)
            t = iet[sl]
            d = idst[sl]
            sv = isrc[sl]
            ia[sl] = d * R + t
            ib[sl] = OFF_B + sv * R + t
            isrc[sl] = OFF_H + sv
            id2[sl] = DEG_BASE + lax.shift_right_logical(d, 7)
            return 0

        lax.fori_loop(0, C // 16, _idx, 0)

        cpa = pltpu.async_copy(tbl_hbm.at[ia], rowa, sem)
        cpb = pltpu.async_copy(tbl_hbm.at[ib], rowb, sem)
        cph = pltpu.async_copy(tbl_hbm.at[isrc], rowh, sem)

        def _oh_set(k, _):
            dv = jnp.bitwise_and(idst[pl.ds(k * 16, 16)], D - 1)
            for i in range(16):
                dl = dv[i]
                off = lax.shift_right_logical(dl, 4) * 16
                lane = jnp.bitwise_and(dl, 15)
                oh_v[k * 16 + i, pl.ds(off, 16)] = jnp.where(
                    iota16 == lane, jnp.float32(1.0), jnp.float32(0.0))
            return 0

        lax.fori_loop(0, C // 16, _oh_set, 0)

        cpa.wait()
        cpb.wait()
        cph.wait()

        def _gate_row(j, _):
            for k in range(D // 16):
                sl = pl.ds(k * 16, 16)
                x = rowa[j, sl] + rowb[j, sl]
                g_ = 1.0 / (1.0 + jnp.exp(-x))
                rowh[j, sl] = rowh[j, sl] * g_
            return 0

        lax.fori_loop(0, C, _gate_row, 0)

        pltpu.sync_copy(rowh, acc.at[idst], add=True)
        pltpu.sync_copy(oh_v, acc.at[id2], add=True)

        def _oh_clr(k, _):
            dv = jnp.bitwise_and(idst[pl.ds(k * 16, 16)], D - 1)
            for i in range(16):
                off = lax.shift_right_logical(dv[i], 4) * 16
                oh_v[k * 16 + i, pl.ds(off, 16)] = zeros16
            return 0

        lax.fori_loop(0, C // 16, _oh_clr, 0)
        return 0

    lax.fori_loop(0, NCHUNK, _chunk, 0)
    plsc.subcore_barrier()

    def _wb(t, _):
        r0 = base_row + t * WB
        pltpu.sync_copy(acc.at[pl.ds(r0, WB)], wb)
        pltpu.sync_copy(wb, out_hbm.at[c, pl.ds(r0, WB)])
        return 0

    lax.fori_loop(0, NB, _wb, 0)


def _edge_stage(tbl, srcp, dstp, etp):
    mesh = plsc.VectorSubcoreMesh(core_axis_name="c", subcore_axis_name="s")
    fn = pl.kernel(
        _edge_body,
        out_type=jax.ShapeDtypeStruct((NC, ACC_ROWS, D), jnp.float32),
        mesh=mesh,
        scratch_types=[
            pltpu.VMEM((C,), jnp.int32),
            pltpu.VMEM((C,), jnp.int32),
            pltpu.VMEM((C,), jnp.int32),
            pltpu.VMEM((C,), jnp.int32),
            pltpu.VMEM((C,), jnp.int32),
            pltpu.VMEM((C,), jnp.int32),
            pltpu.VMEM((C, D), jnp.float32),
            pltpu.VMEM((C, D), jnp.float32),
            pltpu.VMEM((C, D), jnp.float32),
            pltpu.VMEM((C, D), jnp.float32),
            pltpu.VMEM((WB, D), jnp.float32),
            pltpu.VMEM_SHARED((ACC_ROWS, D), jnp.float32),
            pltpu.SemaphoreType.DMA,
        ],
    )
    return fn(tbl, srcp, dstp, etp)


def _final_body(h_ref, m_ref, d_ref, w_ref, b_ref, o_ref):
    sums = m_ref[0] + m_ref[1]
    deg = d_ref[0] + d_ref[1]
    h_n = sums / jnp.maximum(deg, 1.0)
    x = (jnp.dot(h_ref[...], w_ref[:D, :], preferred_element_type=jnp.float32)
         + jnp.dot(h_n, w_ref[D:, :], preferred_element_type=jnp.float32)
         + b_ref[...])
    o_ref[...] = jnp.where(x >= 0, x, x * jnp.float32(0.01))


def _final(h_pad, m_parts, deg_parts, w, b2d):
    bm = NP // 4
    return pl.pallas_call(
        _final_body,
        grid=(4,),
        in_specs=[
            pl.BlockSpec((bm, D), lambda i: (i, 0)),
            pl.BlockSpec((NC, bm, D), lambda i: (0, i, 0)),
            pl.BlockSpec((NC, bm, 1), lambda i: (0, i, 0)),
            pl.BlockSpec((2 * D, OUT), lambda i: (0, 0)),
            pl.BlockSpec((1, OUT), lambda i: (0, 0)),
        ],
        out_specs=pl.BlockSpec((bm, OUT), lambda i: (i, 0)),
        out_shape=jax.ShapeDtypeStruct((NP, OUT), jnp.float32),
    )(h_pad, m_parts, deg_parts, w, b2d)


def kernel(h, edge_index, edge_type, r, W, b):
    rd = jnp.transpose(r[:, :D, :], (1, 0, 2)).reshape(D, R * D)
    rs = jnp.transpose(r[:, D:, :], (1, 0, 2)).reshape(D, R * D)
    h_pad = jnp.concatenate([h, jnp.zeros((NP - N, D), jnp.float32)], axis=0)

    a_arr, b_arr = _project(h_pad, rd, rs)
    tbl = jnp.concatenate(
        [a_arr.reshape(OFF_B, D), b_arr.reshape(OFF_B, D), h_pad], axis=0)

    npad = EP - E
    srcp = jnp.concatenate([edge_index[0], jnp.zeros((npad,), jnp.int32)])
    dstp = jnp.concatenate([edge_index[1], jnp.full((npad,), N, jnp.int32)])
    etp = jnp.concatenate([edge_type, jnp.zeros((npad,), jnp.int32)])

    parts = _edge_stage(tbl, srcp, dstp, etp)
    m_parts = parts[:, :NP, :]
    deg_parts = parts[:, DEG_BASE:DEG_BASE + DEG_ROWS, :].reshape(NC, NP, 1)

    out = _final(h_pad, m_parts, deg_parts, W, b.reshape(1, OUT))
    return out[:N]


# speedup vs baseline: 3.4729x; 1.2447x over previous
import jax
import jax.numpy as jnp
from jax import lax
from jax.experimental import pallas as pl
from jax.experimental.pallas import tpu as pltpu
from jax.experimental.pallas import tpu_sc as plsc

N = 10000
E = 160000
D = 128
R = 4
OUT = 128

NC = 2
NS = 16
NW = NC * NS

NP = 10240
OFF_B = NP * R
OFF_H = 2 * NP * R

DEG_BASE = NP
DEG_ROWS = NP // D
ACC_ROWS = 10368
ROWS_PER_SUB = ACC_ROWS // NS
WB = 24
NB = ROWS_PER_SUB // WB

C = 64
EW = 5120
EP = NW * EW
NCHUNK = EW // C
CW = 3 * C


def _proj_body(h_ref, rd_ref, rs_ref, a_ref, b_ref):
    hb = h_ref[...]
    a_ref[...] = jnp.dot(hb, rd_ref[...], preferred_element_type=jnp.float32)
    b_ref[...] = jnp.dot(hb, rs_ref[...], preferred_element_type=jnp.float32)


def _project(h_pad, rd, rs):
    bm = NP // 4
    return pl.pallas_call(
        _proj_body,
        grid=(4,),
        in_specs=[
            pl.BlockSpec((bm, D), lambda i: (i, 0)),
            pl.BlockSpec((D, R * D), lambda i: (0, 0)),
            pl.BlockSpec((D, R * D), lambda i: (0, 0)),
        ],
        out_specs=[
            pl.BlockSpec((bm, R * D), lambda i: (i, 0)),
            pl.BlockSpec((bm, R * D), lambda i: (i, 0)),
        ],
        out_shape=[
            jax.ShapeDtypeStruct((NP, R * D), jnp.float32),
            jax.ShapeDtypeStruct((NP, R * D), jnp.float32),
        ],
    )(h_pad, rd, rs)


def _edge_body(tbl_hbm, eidx_hbm, out_hbm,
               eidx0, eidx1, ahx0, ahx1, bx0, bx1, idm0, idm1, idd0, idd1,
               ah0, ah1, oh_v, wb, acc, sem_a, sem_b):
    c = lax.axis_index("c")
    s = lax.axis_index("s")
    wid = s * NC + c

    sets = ((eidx0, ahx0, bx0, idm0, idd0, ah0),
            (eidx1, ahx1, bx1, idm1, idd1, ah1))

    zeros16 = jnp.zeros((16,), jnp.float32)
    iota16 = lax.iota(jnp.int32, 16)

    def _zero_wb(i, _):
        for k in range(D // 16):
            wb[i, pl.ds(k * 16, 16)] = zeros16
        return 0

    lax.fori_loop(0, WB, _zero_wb, 0)

    def _zero_oh(j, _):
        for k in range(D // 16):
            oh_v[j, pl.ds(k * 16, 16)] = zeros16
        return 0

    lax.fori_loop(0, C, _zero_oh, 0)

    base_row = s * ROWS_PER_SUB

    def _zero_acc(t, _):
        pltpu.sync_copy(wb, acc.at[pl.ds(base_row + t * WB, WB)])
        return 0

    lax.fori_loop(0, NB, _zero_acc, 0)
    plsc.subcore_barrier()

    wchunk = wid * NCHUNK

    def _prep(chunk, st):
        eidx, ahx, bx, idm, idd, ah = st
        pltpu.sync_copy(eidx_hbm.at[pl.ds(chunk * CW, CW)], eidx)

        def _mkidx(k, _):
            sl = pl.ds(k * 16, 16)
            sv = eidx[pl.ds(k * 16, 16)]
            dv = eidx[pl.ds(C + k * 16, 16)]
            tv = eidx[pl.ds(2 * C + k * 16, 16)]
            ahx[sl] = dv * R + tv
            ahx[pl.ds(C + k * 16, 16)] = OFF_H + sv
            bx[sl] = OFF_B + sv * R + tv
            idm[sl] = dv
            idd[sl] = DEG_BASE + lax.shift_right_logical(dv, 7)
            return 0

        lax.fori_loop(0, C // 16, _mkidx, 0)
        pltpu.async_copy(tbl_hbm.at[ahx], ah, sem_a)

    _prep(wchunk, sets[0])

    def _pair(g2, _):
        for b in (0, 1):
            g = g2 * 2 + b
            eidx, ahx, bx, idm, idd, ah = sets[b]

            pltpu.make_async_copy(tbl_hbm.at[ahx], ah, sem_a).wait()
            cpb = pltpu.async_copy(
                tbl_hbm.at[bx], ah.at[pl.ds(0, C)], sem_b, add=True)

            @pl.when(g + 1 < NCHUNK)
            def _():
                _prep(wchunk + g + 1, sets[1 - b])

            def _oh_set(k, _):
                dv = jnp.bitwise_and(eidx[pl.ds(C + k * 16, 16)], D - 1)
                for i in range(16):
                    dl = dv[i]
                    off = lax.shift_right_logical(dl, 4) * 16
                    lane = jnp.bitwise_and(dl, 15)
                    oh_v[k * 16 + i, pl.ds(off, 16)] = jnp.where(
                        iota16 == lane, jnp.float32(1.0), jnp.float32(0.0))
                return 0

            lax.fori_loop(0, C // 16, _oh_set, 0)

            cpb.wait()

            def _gate_row(j, _):
                for k in range(D // 16):
                    sl = pl.ds(k * 16, 16)
                    x = ah[j, sl]
                    g_ = 1.0 / (1.0 + jnp.exp(-x))
                    ah[C + j, sl] = ah[C + j, sl] * g_
                return 0

            lax.fori_loop(0, C, _gate_row, 0)

            pltpu.sync_copy(ah.at[pl.ds(C, C)], acc.at[idm], add=True)
            pltpu.sync_copy(oh_v, acc.at[idd], add=True)

            def _oh_clr(k, _):
                dv = jnp.bitwise_and(eidx[pl.ds(C + k * 16, 16)], D - 1)
                for i in range(16):
                    off = lax.shift_right_logical(dv[i], 4) * 16
                    oh_v[k * 16 + i, pl.ds(off, 16)] = zeros16
                return 0

            lax.fori_loop(0, C // 16, _oh_clr, 0)
        return 0

    lax.fori_loop(0, NCHUNK // 2, _pair, 0)
    plsc.subcore_barrier()

    def _wb(t, _):
        r0 = base_row + t * WB
        pltpu.sync_copy(acc.at[pl.ds(r0, WB)], wb)
        pltpu.sync_copy(wb, out_hbm.at[c, pl.ds(r0, WB)])
        return 0

    lax.fori_loop(0, NB, _wb, 0)


def _edge_stage(tbl, eidx):
    mesh = plsc.VectorSubcoreMesh(core_axis_name="c", subcore_axis_name="s")
    fn = pl.kernel(
        _edge_body,
        out_type=jax.ShapeDtypeStruct((NC, ACC_ROWS, D), jnp.float32),
        mesh=mesh,
        scratch_types=[
            pltpu.VMEM((CW,), jnp.int32),
            pltpu.VMEM((CW,), jnp.int32),
            pltpu.VMEM((2 * C,), jnp.int32),
            pltpu.VMEM((2 * C,), jnp.int32),
            pltpu.VMEM((C,), jnp.int32),
            pltpu.VMEM((C,), jnp.int32),
            pltpu.VMEM((C,), jnp.int32),
            pltpu.VMEM((C,), jnp.int32),
            pltpu.VMEM((C,), jnp.int32),
            pltpu.VMEM((C,), jnp.int32),
            pltpu.VMEM((2 * C, D), jnp.float32),
            pltpu.VMEM((2 * C, D), jnp.float32),
            pltpu.VMEM((C, D), jnp.float32),
            pltpu.VMEM((WB, D), jnp.float32),
            pltpu.VMEM_SHARED((ACC_ROWS, D), jnp.float32),
            pltpu.SemaphoreType.DMA,
            pltpu.SemaphoreType.DMA,
        ],
    )
    return fn(tbl, eidx)


def _final_body(h_ref, m_ref, d_ref, w_ref, b_ref, o_ref):
    sums = m_ref[0] + m_ref[1]
    deg = d_ref[0] + d_ref[1]
    h_n = sums / jnp.maximum(deg, 1.0)
    x = (jnp.dot(h_ref[...], w_ref[:D, :], preferred_element_type=jnp.float32)
         + jnp.dot(h_n, w_ref[D:, :], preferred_element_type=jnp.float32)
         + b_ref[...])
    o_ref[...] = jnp.where(x >= 0, x, x * jnp.float32(0.01))


def _final(h_pad, m_parts, deg_parts, w, b2d):
    bm = NP // 4
    return pl.pallas_call(
        _final_body,
        grid=(4,),
        in_specs=[
            pl.BlockSpec((bm, D), lambda i: (i, 0)),
            pl.BlockSpec((NC, bm, D), lambda i: (0, i, 0)),
            pl.BlockSpec((NC, bm, 1), lambda i: (0, i, 0)),
            pl.BlockSpec((2 * D, OUT), lambda i: (0, 0)),
            pl.BlockSpec((1, OUT), lambda i: (0, 0)),
        ],
        out_specs=pl.BlockSpec((bm, OUT), lambda i: (i, 0)),
        out_shape=jax.ShapeDtypeStruct((NP, OUT), jnp.float32),
    )(h_pad, m_parts, deg_parts, w, b2d)


def kernel(h, edge_index, edge_type, r, W, b):
    rd = jnp.transpose(r[:, :D, :], (1, 0, 2)).reshape(D, R * D)
    rs = jnp.transpose(r[:, D:, :], (1, 0, 2)).reshape(D, R * D)
    h_pad = jnp.concatenate([h, jnp.zeros((NP - N, D), jnp.float32)], axis=0)

    a_arr, b_arr = _project(h_pad, rd, rs)
    tbl = jnp.concatenate(
        [a_arr.reshape(OFF_B, D), b_arr.reshape(OFF_B, D), h_pad], axis=0)

    npad = EP - E
    srcp = jnp.concatenate([edge_index[0], jnp.zeros((npad,), jnp.int32)])
    dstp = jnp.concatenate([edge_index[1], jnp.full((npad,), N, jnp.int32)])
    etp = jnp.concatenate([edge_type, jnp.zeros((npad,), jnp.int32)])
    eidx = (jnp.stack([srcp, dstp, etp], axis=0)
            .reshape(3, EP // C, C)
            .transpose(1, 0, 2)
            .reshape(3 * EP))

    parts = _edge_stage(tbl, eidx)
    m_parts = parts[:, :NP, :]
    deg_parts = parts[:, DEG_BASE:DEG_BASE + DEG_ROWS, :].reshape(NC, NP, 1)

    out = _final(h_pad, m_parts, deg_parts, W, b.reshape(1, OUT))
    return out[:N]


# speedup vs baseline: 4.1956x; 1.2081x over previous
import jax
import jax.numpy as jnp
from jax import lax
from jax.experimental import pallas as pl
from jax.experimental.pallas import tpu as pltpu
from jax.experimental.pallas import tpu_sc as plsc

N = 10000
E = 160000
D = 128
R = 4
OUT = 128

NC = 2
NS = 16
NW = NC * NS

NP = 10240
OFF_B = R * NP
OFF_H = 2 * R * NP
TBL_ROWS = OFF_H + NP

DEG_BASE = NP
DEG_ROWS = NP // D
ACC_ROWS = 10368
ROWS_PER_SUB = ACC_ROWS // NS
WB = 24
NB = ROWS_PER_SUB // WB

C = 64
EW = 5120
EP = NW * EW
NCHUNK = EW // C
CW = 3 * C


def _proj_body(h_ref, rw_ref, o_ref):
    q = pl.program_id(1)

    @pl.when(q == 2 * R)
    def _():
        o_ref[...] = h_ref[...]

    @pl.when(q != 2 * R)
    def _():
        o_ref[...] = jnp.dot(h_ref[...], rw_ref[0],
                             preferred_element_type=jnp.float32)


def _project(h_pad, rw):
    bm = NP // 4
    return pl.pallas_call(
        _proj_body,
        grid=(4, 2 * R + 1),
        in_specs=[
            pl.BlockSpec((bm, D), lambda i, q: (i, 0)),
            pl.BlockSpec((1, D, OUT), lambda i, q: (q, 0, 0)),
        ],
        out_specs=pl.BlockSpec((bm, OUT), lambda i, q: (q * 4 + i, 0)),
        out_shape=jax.ShapeDtypeStruct((TBL_ROWS, OUT), jnp.float32),
    )(h_pad, rw)


def _edge_body(tbl_hbm, eidx_hbm, out_hbm,
               eidx0, eidx1, ahx0, ahx1, bx0, bx1, idm0, idm1, idd0, idd1,
               ah0, ah1, oh_v, wb, acc, sem_a, sem_b, sem_m, sem_o):
    c = lax.axis_index("c")
    s = lax.axis_index("s")
    wid = s * NC + c

    sets = ((eidx0, ahx0, bx0, idm0, idd0, ah0),
            (eidx1, ahx1, bx1, idm1, idd1, ah1))

    zeros16 = jnp.zeros((16,), jnp.float32)
    iota16 = lax.iota(jnp.int32, 16)

    def _zero_wb(i, _):
        for k in range(D // 16):
            wb[i, pl.ds(k * 16, 16)] = zeros16
        return 0

    lax.fori_loop(0, WB, _zero_wb, 0)

    def _zero_oh(j, _):
        for k in range(D // 16):
            oh_v[j, pl.ds(k * 16, 16)] = zeros16
        return 0

    lax.fori_loop(0, C, _zero_oh, 0)

    base_row = s * ROWS_PER_SUB

    def _zero_acc(t, _):
        pltpu.sync_copy(wb, acc.at[pl.ds(base_row + t * WB, WB)])
        return 0

    lax.fori_loop(0, NB, _zero_acc, 0)
    plsc.subcore_barrier()

    wchunk = wid * NCHUNK

    def _prep(chunk, st, wait_m):
        eidx, ahx, bx, idm, idd, ah = st

        @pl.when(wait_m)
        def _():
            pltpu.make_async_copy(
                ah.at[pl.ds(C, C)], acc.at[idm], sem_m).wait()

        pltpu.sync_copy(eidx_hbm.at[pl.ds(chunk * CW, CW)], eidx)

        def _mkidx(k, _):
            sl = pl.ds(k * 16, 16)
            sv = eidx[pl.ds(k * 16, 16)]
            dv = eidx[pl.ds(C + k * 16, 16)]
            tv = eidx[pl.ds(2 * C + k * 16, 16)]
            ahx[sl] = tv * NP + dv
            ahx[pl.ds(C + k * 16, 16)] = OFF_H + sv
            bx[sl] = OFF_B + tv * NP + sv
            idm[sl] = dv
            idd[sl] = DEG_BASE + lax.shift_right_logical(dv, 7)
            return 0

        lax.fori_loop(0, C // 16, _mkidx, 0)
        pltpu.async_copy(tbl_hbm.at[ahx], ah, sem_a)

    _prep(wchunk, sets[0], jnp.bool_(False))

    def _pair(g2, _):
        for b in (0, 1):
            g = g2 * 2 + b
            eidx, ahx, bx, idm, idd, ah = sets[b]
            peidx = sets[1 - b][0]

            pltpu.make_async_copy(tbl_hbm.at[ahx], ah, sem_a).wait()
            pltpu.async_copy(tbl_hbm.at[bx], ah.at[pl.ds(0, C)], sem_b,
                             add=True)

            @pl.when(g > 0)
            def _():
                pltpu.make_async_copy(
                    oh_v, acc.at[sets[1 - b][4]], sem_o).wait()

                def _oh_clr(k, _):
                    dv = jnp.bitwise_and(peidx[pl.ds(C + k * 16, 16)], D - 1)
                    for i in range(16):
                        off = lax.shift_right_logical(dv[i], 4) * 16
                        oh_v[k * 16 + i, pl.ds(off, 16)] = zeros16
                    return 0

                lax.fori_loop(0, C // 16, _oh_clr, 0)

            @pl.when(g + 1 < NCHUNK)
            def _():
                _prep(wchunk + g + 1, sets[1 - b], g + 1 >= 2)

            def _oh_set(k, _):
                dv = jnp.bitwise_and(eidx[pl.ds(C + k * 16, 16)], D - 1)
                for i in range(16):
                    dl = dv[i]
                    off = lax.shift_right_logical(dl, 4) * 16
                    lane = jnp.bitwise_and(dl, 15)
                    oh_v[k * 16 + i, pl.ds(off, 16)] = jnp.where(
                        iota16 == lane, jnp.float32(1.0), jnp.float32(0.0))
                return 0

            lax.fori_loop(0, C // 16, _oh_set, 0)

            pltpu.make_async_copy(
                tbl_hbm.at[bx], ah.at[pl.ds(0, C)], sem_b).wait()

            def _gate_row(j, _):
                for k in range(D // 16):
                    sl = pl.ds(k * 16, 16)
                    x = ah[j, sl]
                    g_ = 1.0 / (1.0 + jnp.exp(-x))
                    ah[C + j, sl] = ah[C + j, sl] * g_
                return 0

            lax.fori_loop(0, C, _gate_row, 0)

            pltpu.async_copy(ah.at[pl.ds(C, C)], acc.at[idm], sem_m,
                             add=True)
            pltpu.async_copy(oh_v, acc.at[idd], sem_o, add=True)
        return 0

    lax.fori_loop(0, NCHUNK // 2, _pair, 0)

    pltpu.make_async_copy(
        sets[0][5].at[pl.ds(C, C)], acc.at[sets[0][3]], sem_m).wait()
    pltpu.make_async_copy(
        sets[1][5].at[pl.ds(C, C)], acc.at[sets[1][3]], sem_m).wait()
    pltpu.make_async_copy(oh_v, acc.at[sets[1][4]], sem_o).wait()
    plsc.subcore_barrier()

    def _wb(t, _):
        r0 = base_row + t * WB
        pltpu.sync_copy(acc.at[pl.ds(r0, WB)], wb)
        pltpu.sync_copy(wb, out_hbm.at[c, pl.ds(r0, WB)])
        return 0

    lax.fori_loop(0, NB, _wb, 0)


def _edge_stage(tbl, eidx):
    mesh = plsc.VectorSubcoreMesh(core_axis_name="c", subcore_axis_name="s")
    fn = pl.kernel(
        _edge_body,
        out_type=jax.ShapeDtypeStruct((NC, ACC_ROWS, D), jnp.float32),
        mesh=mesh,
        scratch_types=[
            pltpu.VMEM((CW,), jnp.int32),
            pltpu.VMEM((CW,), jnp.int32),
            pltpu.VMEM((2 * C,), jnp.int32),
            pltpu.VMEM((2 * C,), jnp.int32),
            pltpu.VMEM((C,), jnp.int32),
            pltpu.VMEM((C,), jnp.int32),
            pltpu.VMEM((C,), jnp.int32),
            pltpu.VMEM((C,), jnp.int32),
            pltpu.VMEM((C,), jnp.int32),
            pltpu.VMEM((C,), jnp.int32),
            pltpu.VMEM((2 * C, D), jnp.float32),
            pltpu.VMEM((2 * C, D), jnp.float32),
            pltpu.VMEM((C, D), jnp.float32),
            pltpu.VMEM((WB, D), jnp.float32),
            pltpu.VMEM_SHARED((ACC_ROWS, D), jnp.float32),
            pltpu.SemaphoreType.DMA,
            pltpu.SemaphoreType.DMA,
            pltpu.SemaphoreType.DMA,
            pltpu.SemaphoreType.DMA,
        ],
    )
    return fn(tbl, eidx)


def _final_body(h_ref, m_ref, d_ref, w_ref, b_ref, o_ref):
    sums = m_ref[0] + m_ref[1]
    deg = d_ref[0] + d_ref[1]
    h_n = sums / jnp.maximum(deg, 1.0)
    x = (jnp.dot(h_ref[...], w_ref[:D, :], preferred_element_type=jnp.float32)
         + jnp.dot(h_n, w_ref[D:, :], preferred_element_type=jnp.float32)
         + b_ref[...])
    o_ref[...] = jnp.where(x >= 0, x, x * jnp.float32(0.01))


def _final(h_pad, parts, deg_parts, w, b2d):
    bm = NP // 4
    return pl.pallas_call(
        _final_body,
        grid=(4,),
        in_specs=[
            pl.BlockSpec((bm, D), lambda i: (i, 0)),
            pl.BlockSpec((NC, bm, D), lambda i: (0, i, 0)),
            pl.BlockSpec((NC, bm, 1), lambda i: (0, i, 0)),
            pl.BlockSpec((2 * D, OUT), lambda i: (0, 0)),
            pl.BlockSpec((1, OUT), lambda i: (0, 0)),
        ],
        out_specs=pl.BlockSpec((bm, OUT), lambda i: (i, 0)),
        out_shape=jax.ShapeDtypeStruct((NP, OUT), jnp.float32),
    )(h_pad, parts, deg_parts, w, b2d)


def kernel(h, edge_index, edge_type, r, W, b):
    rw = jnp.concatenate(
        [r[:, :D, :], r[:, D:, :], jnp.eye(D, OUT, dtype=jnp.float32)[None]],
        axis=0)
    h_pad = jnp.concatenate([h, jnp.zeros((NP - N, D), jnp.float32)], axis=0)

    tbl = _project(h_pad, rw)

    npad = EP - E
    srcp = jnp.concatenate([edge_index[0], jnp.zeros((npad,), jnp.int32)])
    dstp = jnp.concatenate([edge_index[1], jnp.full((npad,), N, jnp.int32)])
    etp = jnp.concatenate([edge_type, jnp.zeros((npad,), jnp.int32)])
    eidx = (jnp.stack([srcp, dstp, etp], axis=0)
            .reshape(3, EP // C, C)
            .transpose(1, 0, 2)
            .reshape(3 * EP))

    parts = _edge_stage(tbl, eidx)
    deg_parts = parts[:, DEG_BASE:DEG_BASE + DEG_ROWS, :].reshape(NC, NP, 1)

    out = _final(h_pad, parts, deg_parts, W, b.reshape(1, OUT))
    return out[:N]
